# baseline (speedup 1.0000x reference)
import functools

import jax
import jax.numpy as jnp
from jax import lax
from jax.experimental import pallas as pl
from jax.experimental.pallas import tpu as pltpu
from jax.experimental.pallas import tpu_sc as plsc

T = 16
B, S, D = 2, 2048, 1024
NW = 32
R = 2
CW = R * D
ROWS_PER_W = (B * S) // NW
CHUNKS = ROWS_PER_W // R
VPC = CW // 16


def _sc_body(x_hbm, out_hbm, xbuf0, xbuf1, ob0, ob1, st0, st1, sem0, sem1):
    wid = lax.axis_index("s") * 2 + lax.axis_index("c")
    row0 = wid * ROWS_PER_W

    iota = lax.iota(jnp.int32, 16)
    ones = jnp.full((16,), 1.0, jnp.float32)
    zeros = jnp.zeros((16,), jnp.float32)

    xbufs = (xbuf0, xbuf1)
    obufs = (ob0, ob1)
    stbufs = (st0, st1)
    sems = (sem0, sem1)

    def _zero(i, _):
        ob0[pl.ds(i * 16, 16)] = zeros
        ob1[pl.ds(i * 16, 16)] = zeros
        return 0
    lax.fori_loop(0, T * CW // 16, _zero, 0)

    def outer(c2, _):
        for slot in range(2):
            xbuf, obuf, stbuf, sem = xbufs[slot], obufs[slot], stbufs[slot], sems[slot]
            c = c2 * 2 + slot
            n0 = row0 + c * R
            b = n0 >> 11
            s0 = n0 & 2047

            @pl.when(c2 >= 1)
            def _drain_and_reset():
                pltpu.make_async_copy(
                    x_hbm.at[pl.ds(0, T * CW)], obuf, sem
                ).wait()

                def _reset(i, _):
                    stv = stbuf[pl.ds(i * 16, 16)]
                    idx = stv * CW + i * 16 + iota
                    plsc.store_scatter(obuf, [idx], zeros)
                    return 0
                lax.fori_loop(0, VPC, _reset, 0)

            pltpu.sync_copy(x_hbm.at[pl.ds(n0 * D, CW)], xbuf)

            def _encode(i, _):
                xv = xbuf[pl.ds(i * 16, 16)]
                e = jnp.exp(-jnp.abs(xv))
                sig = jnp.where(xv >= 0.0, 1.0, e) / (1.0 + e)
                stv = (sig * 15.0).astype(jnp.int32)
                idx = stv * CW + i * 16 + iota
                plsc.store_scatter(obuf, [idx], ones)
                stbuf[pl.ds(i * 16, 16)] = stv
                return 0
            lax.fori_loop(0, VPC, _encode, 0)

            out_base = b * (T * S * D) + s0 * D
            for t in range(T):
                pltpu.async_copy(
                    obuf.at[pl.ds(t * CW, CW)],
                    out_hbm.at[pl.ds(out_base + t * (S * D), CW)],
                    sem,
                )
        return 0

    lax.fori_loop(0, CHUNKS // 2, outer, 0)

    for slot in range(2):
        pltpu.make_async_copy(
            x_hbm.at[pl.ds(0, T * CW)], obufs[slot], sems[slot]
        ).wait()


@jax.jit
def _sc_encode(xf):
    k = functools.partial(
        pl.kernel,
        out_type=jax.ShapeDtypeStruct((B * T * S * D,), jnp.float32),
        mesh=plsc.VectorSubcoreMesh(core_axis_name="c", subcore_axis_name="s"),
        compiler_params=pltpu.CompilerParams(needs_layout_passes=False),
        scratch_types=[
            pltpu.VMEM((CW,), jnp.float32),
            pltpu.VMEM((CW,), jnp.float32),
            pltpu.VMEM((T * CW,), jnp.float32),
            pltpu.VMEM((T * CW,), jnp.float32),
            pltpu.VMEM((CW,), jnp.int32),
            pltpu.VMEM((CW,), jnp.int32),
            pltpu.SemaphoreType.DMA,
            pltpu.SemaphoreType.DMA,
        ],
    )(_sc_body)
    return k(xf)


def kernel(x):
    xf = x.reshape(-1)
    out = _sc_encode(xf)
    return out.reshape(B, T, S, D)


# speedup vs baseline: 1.6398x; 1.6398x over previous
import functools

import jax
import jax.numpy as jnp
from jax import lax
from jax.experimental import pallas as pl
from jax.experimental.pallas import tpu as pltpu
from jax.experimental.pallas import tpu_sc as plsc

T = 16
B, S, D = 2, 2048, 1024
NW = 32
R = 2
CW = R * D
ROWS_PER_W = (B * S) // NW
CHUNKS = ROWS_PER_W // R
VPC = CW // 16


def _sc_body(x_hbm, out_hbm, xbuf0, xbuf1, ob0, ob1, st0, st1,
             isem0, isem1, osem0, osem1):
    wid = lax.axis_index("s") * 2 + lax.axis_index("c")
    row0 = wid * ROWS_PER_W

    iota = lax.iota(jnp.int32, 16)
    ones = jnp.full((16,), 1.0, jnp.float32)
    zeros = jnp.zeros((16,), jnp.float32)
    izeros = jnp.zeros((16,), jnp.int32)

    xbufs = (xbuf0, xbuf1)
    obufs = (ob0, ob1)
    stbufs = (st0, st1)
    isems = (isem0, isem1)
    osems = (osem0, osem1)

    @plsc.parallel_loop(0, T * CW // 16, unroll=4)
    def _zero(i):
        ob0[pl.ds(i * 16, 16)] = zeros
        ob1[pl.ds(i * 16, 16)] = zeros

    @plsc.parallel_loop(0, VPC, unroll=4)
    def _zero_st(i):
        st0[pl.ds(i * 16, 16)] = izeros
        st1[pl.ds(i * 16, 16)] = izeros

    for slot in range(2):
        pltpu.async_copy(
            x_hbm.at[pl.ds((row0 + slot * R) * D, CW)], xbufs[slot], isems[slot]
        )

    def outer(c2, _):
        for slot in range(2):
            xbuf, obuf, stbuf = xbufs[slot], obufs[slot], stbufs[slot]
            isem, osem = isems[slot], osems[slot]
            c = c2 * 2 + slot
            n0 = row0 + c * R
            b = n0 >> 11
            s0 = n0 & 2047

            pltpu.make_async_copy(x_hbm.at[pl.ds(0, CW)], xbuf, isem).wait()

            @pl.when(c2 >= 1)
            def _drain_out():
                pltpu.make_async_copy(
                    x_hbm.at[pl.ds(0, T * CW)], obuf, osem
                ).wait()

            @plsc.parallel_loop(0, VPC, unroll=8)
            def _encode(i):
                pos = i * 16 + iota
                xv = xbuf[pl.ds(i * 16, 16)]
                e = jnp.exp(-jnp.abs(xv))
                sig = jnp.where(xv >= 0.0, 1.0, e) / (1.0 + e)
                stv = (sig * 15.0).astype(jnp.int32)
                old = stbuf[pl.ds(i * 16, 16)]
                clear = jnp.where(old == stv, 1.0, 0.0)
                plsc.store_scatter(obuf, [(old << 11) + pos], clear)
                plsc.store_scatter(obuf, [(stv << 11) + pos], ones)
                stbuf[pl.ds(i * 16, 16)] = stv

            out_base = b * (T * S * D) + s0 * D
            for t in range(T):
                pltpu.async_copy(
                    obuf.at[pl.ds(t * CW, CW)],
                    out_hbm.at[pl.ds(out_base + t * (S * D), CW)],
                    osem,
                )

            @pl.when(c2 < CHUNKS // 2 - 1)
            def _prefetch():
                pltpu.async_copy(
                    x_hbm.at[pl.ds((n0 + 2 * R) * D, CW)], xbuf, isem
                )
        return 0

    lax.fori_loop(0, CHUNKS // 2, outer, 0)

    for slot in range(2):
        pltpu.make_async_copy(
            x_hbm.at[pl.ds(0, T * CW)], obufs[slot], osems[slot]
        ).wait()


@jax.jit
def _sc_encode(xf):
    k = functools.partial(
        pl.kernel,
        out_type=jax.ShapeDtypeStruct((B * T * S * D,), jnp.float32),
        mesh=plsc.VectorSubcoreMesh(core_axis_name="c", subcore_axis_name="s"),
        compiler_params=pltpu.CompilerParams(needs_layout_passes=False),
        scratch_types=[
            pltpu.VMEM((CW,), jnp.float32),
            pltpu.VMEM((CW,), jnp.float32),
            pltpu.VMEM((T * CW,), jnp.float32),
            pltpu.VMEM((T * CW,), jnp.float32),
            pltpu.VMEM((CW,), jnp.int32),
            pltpu.VMEM((CW,), jnp.int32),
            pltpu.SemaphoreType.DMA,
            pltpu.SemaphoreType.DMA,
            pltpu.SemaphoreType.DMA,
            pltpu.SemaphoreType.DMA,
        ],
    )(_sc_body)
    return k(xf)


def kernel(x):
    xf = x.reshape(-1)
    out = _sc_encode(xf)
    return out.reshape(B, T, S, D)


# speedup vs baseline: 2.0273x; 1.2363x over previous
import functools

import jax
import jax.numpy as jnp
from jax import lax
from jax.experimental import pallas as pl
from jax.experimental.pallas import tpu as pltpu
from jax.experimental.pallas import tpu_sc as plsc

T = 16
B, S, D = 2, 2048, 1024
NW = 32
R = 2
CW = R * D
ROWS_PER_W = (B * S) // NW
CHUNKS = ROWS_PER_W // R
VPC = CW // 16


def _sc_body(x_hbm, out_hbm, xbuf0, xbuf1, ob0, ob1, st0, st1,
             isem0, isem1, osem0, osem1):
    wid = lax.axis_index("s") * 2 + lax.axis_index("c")
    row0 = wid * ROWS_PER_W

    iota = lax.iota(jnp.int32, 16)
    ones = jnp.full((16,), 1.0, jnp.float32)
    zeros = jnp.zeros((16,), jnp.float32)
    izeros = jnp.zeros((16,), jnp.int32)

    xbufs = (xbuf0, xbuf1)
    obufs = (ob0, ob1)
    stbufs = (st0, st1)
    isems = (isem0, isem1)
    osems = (osem0, osem1)

    for t in range(T):
        @plsc.parallel_loop(0, VPC, unroll=4)
        def _zero(i, t=t):
            ob0[t, pl.ds(i * 16, 16)] = zeros
            ob1[t, pl.ds(i * 16, 16)] = zeros

    @plsc.parallel_loop(0, VPC, unroll=4)
    def _zero_st(i):
        st0[pl.ds(i * 16, 16)] = izeros
        st1[pl.ds(i * 16, 16)] = izeros

    for slot in range(2):
        pltpu.async_copy(
            x_hbm.at[pl.ds((row0 + slot * R) * D, CW)], xbufs[slot], isems[slot]
        )

    def outer(c2, _):
        for slot in range(2):
            xbuf, obuf, stbuf = xbufs[slot], obufs[slot], stbufs[slot]
            isem, osem = isems[slot], osems[slot]
            c = c2 * 2 + slot
            n0 = row0 + c * R
            b = n0 >> 11
            s0 = n0 & 2047

            pltpu.make_async_copy(x_hbm.at[pl.ds(0, CW)], xbuf, isem).wait()

            @pl.when(c2 >= 1)
            def _drain_out():
                pltpu.make_async_copy(
                    out_hbm.at[pl.ds(0, T), pl.ds(0, CW)], obuf, osem
                ).wait()

            @plsc.parallel_loop(0, VPC, unroll=8)
            def _encode(i):
                pos = i * 16 + iota
                xv = xbuf[pl.ds(i * 16, 16)]
                e = jnp.exp(-jnp.abs(xv))
                sig = jnp.where(xv >= 0.0, 1.0, e) / (1.0 + e)
                stv = (sig * 15.0).astype(jnp.int32)
                old = stbuf[pl.ds(i * 16, 16)]
                clear = jnp.where(old == stv, 1.0, 0.0)
                plsc.store_scatter(obuf, [old, pos], clear)
                plsc.store_scatter(obuf, [stv, pos], ones)
                stbuf[pl.ds(i * 16, 16)] = stv

            pltpu.async_copy(
                obuf,
                out_hbm.at[pl.ds(b * T, T), pl.ds(s0 * D, CW)],
                osem,
            )

            @pl.when(c2 < CHUNKS // 2 - 1)
            def _prefetch():
                pltpu.async_copy(
                    x_hbm.at[pl.ds((n0 + 2 * R) * D, CW)], xbuf, isem
                )
        return 0

    lax.fori_loop(0, CHUNKS // 2, outer, 0)

    for slot in range(2):
        pltpu.make_async_copy(
            out_hbm.at[pl.ds(0, T), pl.ds(0, CW)], obufs[slot], osems[slot]
        ).wait()


@jax.jit
def _sc_encode(xf):
    k = functools.partial(
        pl.kernel,
        out_type=jax.ShapeDtypeStruct((B * T, S * D), jnp.float32),
        mesh=plsc.VectorSubcoreMesh(core_axis_name="c", subcore_axis_name="s"),
        compiler_params=pltpu.CompilerParams(needs_layout_passes=False),
        scratch_types=[
            pltpu.VMEM((CW,), jnp.float32),
            pltpu.VMEM((CW,), jnp.float32),
            pltpu.VMEM((T, CW), jnp.float32),
            pltpu.VMEM((T, CW), jnp.float32),
            pltpu.VMEM((CW,), jnp.int32),
            pltpu.VMEM((CW,), jnp.int32),
            pltpu.SemaphoreType.DMA,
            pltpu.SemaphoreType.DMA,
            pltpu.SemaphoreType.DMA,
            pltpu.SemaphoreType.DMA,
        ],
    )(_sc_body)
    return k(xf)


def kernel(x):
    xf = x.reshape(-1)
    out = _sc_encode(xf)
    return out.reshape(B, T, S, D)


# speedup vs baseline: 2.1221x; 1.0468x over previous
import functools

import jax
import jax.numpy as jnp
from jax import lax
from jax.experimental import pallas as pl
from jax.experimental.pallas import tpu as pltpu
from jax.experimental.pallas import tpu_sc as plsc

T = 16
B, S, D = 2, 2048, 1024
NW = 32
R = 2
CW = R * D
ROWS_PER_W = (B * S) // NW
CHUNKS = ROWS_PER_W // R
VPC = CW // 16


def _sc_body(x_hbm, out_hbm, xbuf0, xbuf1, ob0, ob1, st0, st1,
             isem0, isem1, osem0, osem1):
    wid = lax.axis_index("s") * 2 + lax.axis_index("c")
    row0 = wid * ROWS_PER_W

    iota = lax.iota(jnp.int32, 16)
    ones = jnp.full((16,), 1.0, jnp.float32)
    zeros = jnp.zeros((16,), jnp.float32)
    izeros = jnp.zeros((16,), jnp.int32)

    xbufs = (xbuf0, xbuf1)
    obufs = (ob0, ob1)
    stbufs = (st0, st1)
    isems = (isem0, isem1)
    osems = (osem0, osem1)

    for t in range(T):
        @plsc.parallel_loop(0, VPC, unroll=4)
        def _zero(i, t=t):
            ob0[t, pl.ds(i * 16, 16)] = zeros
            ob1[t, pl.ds(i * 16, 16)] = zeros

    @plsc.parallel_loop(0, VPC, unroll=4)
    def _zero_st(i):
        st0[pl.ds(i * 16, 16)] = izeros
        st1[pl.ds(i * 16, 16)] = izeros

    for slot in range(2):
        pltpu.async_copy(
            x_hbm.at[pl.ds((row0 + slot * R) * D, CW)], xbufs[slot], isems[slot]
        )

    def outer(c2, _):
        for slot in range(2):
            xbuf, obuf, stbuf = xbufs[slot], obufs[slot], stbufs[slot]
            isem, osem = isems[slot], osems[slot]
            c = c2 * 2 + slot
            n0 = row0 + c * R
            b = n0 >> 11
            s0 = n0 & 2047

            pltpu.make_async_copy(x_hbm.at[pl.ds(0, CW)], xbuf, isem).wait()

            @pl.when(c2 >= 1)
            def _drain_out():
                pltpu.make_async_copy(
                    out_hbm.at[pl.ds(0, T), pl.ds(0, CW)], obuf, osem
                ).wait()

            @plsc.parallel_loop(0, VPC, unroll=8)
            def _encode(i):
                pos = i * 16 + iota
                xv = xbuf[pl.ds(i * 16, 16)]
                e = jnp.exp(-jnp.abs(xv))
                sig = jnp.where(xv >= 0.0, 1.0, e) / (1.0 + e)
                stv = (sig * 15.0).astype(jnp.int32)
                old = stbuf[pl.ds(i * 16, 16)]
                clear = jnp.where(old == stv, 1.0, 0.0)
                plsc.store_scatter(obuf, [old, pos], clear)
                plsc.store_scatter(obuf, [stv, pos], ones)
                stbuf[pl.ds(i * 16, 16)] = stv

            pltpu.async_copy(
                obuf,
                out_hbm.at[pl.ds(b * T, T), pl.ds(s0 * D, CW)],
                osem,
            )

            @pl.when(c2 < CHUNKS // 2 - 1)
            def _prefetch():
                pltpu.async_copy(
                    x_hbm.at[pl.ds((n0 + 2 * R) * D, CW)], xbuf, isem
                )
        return 0

    lax.fori_loop(0, CHUNKS // 2, outer, 0)

    for slot in range(2):
        pltpu.make_async_copy(
            out_hbm.at[pl.ds(0, T), pl.ds(0, CW)], obufs[slot], osems[slot]
        ).wait()


@jax.jit
def _sc_encode(xf):
    k = functools.partial(
        pl.kernel,
        out_type=jax.ShapeDtypeStruct((B * T, S * D), jnp.float32),
        mesh=plsc.VectorSubcoreMesh(core_axis_name="c", subcore_axis_name="s"),
        compiler_params=pltpu.CompilerParams(needs_layout_passes=False),
        scratch_types=[
            pltpu.VMEM((CW,), jnp.float32),
            pltpu.VMEM((CW,), jnp.float32),
            pltpu.VMEM((T, CW), jnp.float32),
            pltpu.VMEM((T, CW), jnp.float32),
            pltpu.VMEM((CW,), jnp.int32),
            pltpu.VMEM((CW,), jnp.int32),
            pltpu.SemaphoreType.DMA,
            pltpu.SemaphoreType.DMA,
            pltpu.SemaphoreType.DMA,
            pltpu.SemaphoreType.DMA,
        ],
    )(_sc_body)
    return k(xf)


def kernel(x):
    xf = (
        x.reshape(B, S // 8, 8, D // 128, 128)
        .transpose(0, 1, 3, 2, 4)
        .reshape(-1)
    )
    out = _sc_encode(xf)
    return (
        out.reshape(B, T, S // 8, D // 128, 8, 128)
        .transpose(0, 1, 2, 4, 3, 5)
        .reshape(B, T, S, D)
    )


# speedup vs baseline: 5.3688x; 2.5300x over previous
import functools

import jax
import jax.numpy as jnp
from jax import lax
from jax.experimental import pallas as pl
from jax.experimental.pallas import tpu as pltpu
from jax.experimental.pallas import tpu_sc as plsc

T = 16
B, S, D = 2, 2048, 1024
NW = 32
R = 2
CW = R * D
ROWS_PER_W = (B * S) // NW
CHUNKS = ROWS_PER_W // R
VPC = CW // 16


def _sc_body(x_hbm, out_hbm, xbuf0, xbuf1, ob0, ob1, st0, st1,
             isem0, isem1, osem0, osem1):
    wid = lax.axis_index("s") * 2 + lax.axis_index("c")
    row0 = wid * ROWS_PER_W

    iota = lax.iota(jnp.int32, 16)
    ones = jnp.full((16,), 1.0, jnp.float32)
    zeros = jnp.zeros((16,), jnp.float32)
    izeros = jnp.zeros((16,), jnp.int32)

    xbufs = (xbuf0, xbuf1)
    obufs = (ob0, ob1)
    stbufs = (st0, st1)
    isems = (isem0, isem1)
    osems = (osem0, osem1)

    @plsc.parallel_loop(0, T * CW // 16, unroll=4)
    def _zero(i):
        ob0[pl.ds(i * 16, 16)] = zeros
        ob1[pl.ds(i * 16, 16)] = zeros

    @plsc.parallel_loop(0, VPC, unroll=4)
    def _zero_st(i):
        st0[pl.ds(i * 16, 16)] = izeros
        st1[pl.ds(i * 16, 16)] = izeros

    for slot in range(2):
        pltpu.async_copy(
            x_hbm.at[pl.ds((row0 + slot * R) * D, CW)], xbufs[slot], isems[slot]
        )

    def outer(c2, _):
        for slot in range(2):
            xbuf, obuf, stbuf = xbufs[slot], obufs[slot], stbufs[slot]
            isem, osem = isems[slot], osems[slot]
            c = c2 * 2 + slot
            n0 = row0 + c * R
            b = n0 >> 11
            s0 = n0 & 2047

            pltpu.make_async_copy(x_hbm.at[pl.ds(0, CW)], xbuf, isem).wait()

            @pl.when(c2 >= 1)
            def _drain_out():
                pltpu.make_async_copy(
                    out_hbm.at[pl.ds(0, T * CW)], obuf, osem
                ).wait()

            @plsc.parallel_loop(0, VPC, unroll=8)
            def _encode(i):
                pos = i * 16 + iota
                xv = xbuf[pl.ds(i * 16, 16)]
                e = jnp.exp(-jnp.abs(xv))
                sig = jnp.where(xv >= 0.0, 1.0, e) / (1.0 + e)
                stv = (sig * 15.0).astype(jnp.int32)
                old = stbuf[pl.ds(i * 16, 16)]
                clear = jnp.where(old == stv, 1.0, 0.0)
                plsc.store_scatter(obuf, [(old << 11) + pos], clear)
                plsc.store_scatter(obuf, [(stv << 11) + pos], ones)
                stbuf[pl.ds(i * 16, 16)] = stv

            out_base = b * (T * S * D) + s0 * D
            for t_ in range(T):
                pltpu.async_copy(
                    obuf.at[pl.ds(t_ * CW, CW)],
                    out_hbm.at[pl.ds(out_base + t_ * (S * D), CW)],
                    osem,
                )

            @pl.when(c2 < CHUNKS // 2 - 1)
            def _prefetch():
                pltpu.async_copy(
                    x_hbm.at[pl.ds((n0 + 2 * R) * D, CW)], xbuf, isem
                )
        return 0

    lax.fori_loop(0, CHUNKS // 2, outer, 0)

    for slot in range(2):
        pltpu.make_async_copy(
            out_hbm.at[pl.ds(0, T * CW)], obufs[slot], osems[slot]
        ).wait()


@jax.jit
def _sc_encode(xf):
    k = functools.partial(
        pl.kernel,
        out_type=jax.ShapeDtypeStruct((B * T * S * D,), jnp.float32),
        mesh=plsc.VectorSubcoreMesh(core_axis_name="c", subcore_axis_name="s"),
        compiler_params=pltpu.CompilerParams(needs_layout_passes=False),
        scratch_types=[
            pltpu.VMEM((CW,), jnp.float32),
            pltpu.VMEM((CW,), jnp.float32),
            pltpu.VMEM((T * CW,), jnp.float32),
            pltpu.VMEM((T * CW,), jnp.float32),
            pltpu.VMEM((CW,), jnp.int32),
            pltpu.VMEM((CW,), jnp.int32),
            pltpu.SemaphoreType.DMA,
            pltpu.SemaphoreType.DMA,
            pltpu.SemaphoreType.DMA,
            pltpu.SemaphoreType.DMA,
        ],
    )(_sc_body)
    return k(xf)


def kernel(x):
    xf = (
        x.reshape(B, S // 8, 8, D // 128, 128)
        .transpose(0, 1, 3, 2, 4)
        .reshape(-1)
    )
    out = _sc_encode(xf)
    return (
        out.reshape(B, T, S // 8, D // 128, 8, 128)
        .transpose(0, 1, 2, 4, 3, 5)
        .reshape(B, T, S, D)
    )
